# baseline (speedup 1.0000x reference)
import functools

import jax
import jax.numpy as jnp
from jax import lax
from jax.experimental import pallas as pl
from jax.experimental.pallas import tpu as pltpu
from jax.experimental.pallas import tpu_sc as plsc

N_NODES = 10000
D_FEAT = 128
C = 16
NC = 2
NS = 16
NW = NC * NS
N_PAD = 10240
RPS = N_PAD // NS
RPW = N_PAD // NW
E = 320000
CHUNK = 128
K_BUF = 8
LAG = 4
CH = 80
E_PAD = NW * CH * CHUNK

_MESH = plsc.VectorSubcoreMesh(core_axis_name="c", subcore_axis_name="s")


def _worker_id():
    return lax.axis_index("s") * NC + lax.axis_index("c")


_MM_BLK = 2048


def _mm_body(x_ref, w_ref, d0_ref, d1_ref, g_ref, dis_ref, dis2_ref):
    deg = d0_ref[0] + d1_ref[0]
    dis = lax.rsqrt(deg)
    dis_ref[...] = dis
    dis2_ref[...] = 1.0 / deg
    y = jnp.dot(x_ref[...], w_ref[...], preferred_element_type=jnp.float32)
    g_ref[...] = dis * y


def _matmul_scale1(x_pad, wt, dparts):
    d3 = dparts.reshape(NC, N_PAD, C)
    return pl.pallas_call(
        _mm_body,
        grid=(N_PAD // _MM_BLK,),
        in_specs=[
            pl.BlockSpec((_MM_BLK, D_FEAT), lambda i: (i, 0)),
            pl.BlockSpec((D_FEAT, C), lambda i: (0, 0)),
            pl.BlockSpec((1, _MM_BLK, C), lambda i: (0, i, 0)),
            pl.BlockSpec((1, _MM_BLK, C), lambda i: (1, i, 0)),
        ],
        out_specs=[
            pl.BlockSpec((_MM_BLK, C), lambda i: (i, 0)),
            pl.BlockSpec((_MM_BLK, C), lambda i: (i, 0)),
            pl.BlockSpec((_MM_BLK, C), lambda i: (i, 0)),
        ],
        out_shape=[jax.ShapeDtypeStruct((N_PAD, C), jnp.float32)] * 3,
    )(x_pad, wt, d3, d3)


def _make_prop(mode):
    do_gather = mode != "deg"
    scratch = [
        pltpu.VMEM_SHARED((N_PAD, C), jnp.float32),
        pltpu.VMEM((CH, CHUNK), jnp.int32),
        pltpu.VMEM((K_BUF, CHUNK, C), jnp.float32),
        pltpu.SemaphoreType.DMA((K_BUF,)),
    ]
    if do_gather:
        scratch += [
            pltpu.VMEM_SHARED((N_PAD, C), jnp.float32),
            pltpu.VMEM((CH, CHUNK), jnp.int32),
            pltpu.SemaphoreType.DMA((K_BUF,)),
        ]
    if mode == "mid":
        scratch += [pltpu.VMEM((RPS, C), jnp.float32) for _ in range(4)]

    def body(g_hbm, aux_hbm, src_hbm, dst_hbm, z_hbm, out_hbm, S, dstv, rows,
             ssem, *rest):
        c = lax.axis_index("c")
        s = lax.axis_index("s")
        wid = _worker_id()
        pltpu.sync_copy(dst_hbm.at[wid], dstv)
        sl = pl.ds(s * RPS, RPS)
        if do_gather:
            G, srcv, gsem = rest[:3]
            pltpu.sync_copy(src_hbm.at[wid], srcv)
        if mode == "prop":
            G = g_hbm
        elif mode == "deg":
            def fill(i, carry):
                rows[0, i] = jnp.ones((C,), jnp.float32)
                return carry
            lax.fori_loop(0, CHUNK, fill, 0)
        elif mode == "mid":
            p0v, p1v, d2v, gv = rest[3:]
            pltpu.sync_copy(g_hbm.at[pl.ds(s * RPS, RPS)], p0v)
            pltpu.sync_copy(g_hbm.at[pl.ds(N_PAD + s * RPS, RPS)], p1v)
            pltpu.sync_copy(aux_hbm.at[sl], d2v)

            def srow(i, carry):
                gv[i] = (p0v[i] + p1v[i]) * d2v[i]
                return carry

            lax.fori_loop(0, RPS, srow, 0)
            pltpu.sync_copy(gv, G.at[sl])

        @pl.when(c == 0)
        def _():
            if mode == "mid":
                pltpu.sync_copy(gv, S.at[sl])
            else:
                pltpu.sync_copy(g_hbm.at[sl], S.at[sl])

        @pl.when(c != 0)
        def _():
            pltpu.sync_copy(z_hbm.at[sl], S.at[sl])

        plsc.subcore_barrier()

        if do_gather:
            gdesc = [None] * CH
            sdesc = [None] * CH

            def issue_scatter(j):
                gdesc[j].wait()
                sdesc[j] = pltpu.async_copy(
                    rows.at[j % K_BUF], S.at[dstv.at[j]],
                    ssem.at[j % K_BUF], add=True)

            for j in range(CH):
                if j >= K_BUF:
                    sdesc[j - K_BUF].wait()
                gdesc[j] = pltpu.async_copy(G.at[srcv.at[j]],
                                            rows.at[j % K_BUF],
                                            gsem.at[j % K_BUF])
                if j >= LAG:
                    issue_scatter(j - LAG)
            for j in range(CH - LAG, CH):
                issue_scatter(j)
            for j in range(CH - K_BUF, CH):
                sdesc[j].wait()
        else:
            sdesc = [None] * CH
            for j in range(CH):
                if j >= K_BUF:
                    sdesc[j - K_BUF].wait()
                sdesc[j] = pltpu.async_copy(rows.at[0], S.at[dstv.at[j]],
                                            ssem.at[j % K_BUF], add=True)
            for j in range(CH - K_BUF, CH):
                sdesc[j].wait()
        plsc.subcore_barrier()
        pltpu.sync_copy(S.at[sl], out_hbm.at[pl.ds(c * N_PAD + s * RPS, RPS)])

    return pl.kernel(
        body,
        out_type=jax.ShapeDtypeStruct((NC * N_PAD, C), jnp.float32),
        mesh=_MESH,
        scratch_types=scratch,
        compiler_params=pltpu.CompilerParams(use_tc_tiling_on_sc=False),
    )


_prop_deg = _make_prop("deg")
_prop_gather = _make_prop("prop")
_prop_mid = _make_prop("mid")


def _scale3_body(parts_ref, dis_ref, b_ref, o_ref):
    s = parts_ref[:N_PAD, :] + parts_ref[N_PAD:, :]
    o_ref[...] = dis_ref[...] * s + b_ref[...]


def _scale3(parts, dis, b):
    return pl.pallas_call(
        _scale3_body,
        out_shape=jax.ShapeDtypeStruct((N_PAD, C), jnp.float32),
    )(parts, dis, b.reshape(1, C))


def kernel(x, edge_index, W, b):
    src = edge_index[0].astype(jnp.int32)
    dst = edge_index[1].astype(jnp.int32)
    pad = E_PAD - E
    dummy = jnp.full((pad,), N_NODES, jnp.int32)
    src3 = jnp.concatenate([src, dummy]).reshape(NW, CH, CHUNK)
    dst3 = jnp.concatenate([dst, dummy]).reshape(NW, CH, CHUNK)

    x_pad = jnp.pad(x, ((0, N_PAD - N_NODES), (0, 0)))

    ones_tab = jnp.ones((N_PAD, C), jnp.float32)
    zeros_tab = jnp.zeros((N_PAD, C), jnp.float32)

    deg_parts = _prop_deg(ones_tab, zeros_tab, src3, dst3, zeros_tab)
    g1, dis, dis2 = _matmul_scale1(x_pad, W.T, deg_parts)
    p1 = _prop_gather(g1, zeros_tab, src3, dst3, zeros_tab)
    p2 = _prop_mid(p1, dis2, src3, dst3, zeros_tab)
    out_pad = _scale3(p2, dis, b)
    return out_pad[:N_NODES]


# speedup vs baseline: 1.2584x; 1.2584x over previous
import functools

import jax
import jax.numpy as jnp
from jax import lax
from jax.experimental import pallas as pl
from jax.experimental.pallas import tpu as pltpu
from jax.experimental.pallas import tpu_sc as plsc

N_NODES = 10000
D_FEAT = 128
C = 16
NC = 2
NS = 16
NW = NC * NS
N_PAD = 10240
RPS = N_PAD // NS
RPW = N_PAD // NW
E = 320000
CHUNK = 128
K_BUF = 8
LAG = 4
CH = 80
E_PAD = NW * CH * CHUNK

_MESH = plsc.VectorSubcoreMesh(core_axis_name="c", subcore_axis_name="s")


def _worker_id():
    return lax.axis_index("s") * NC + lax.axis_index("c")


_MM_BLK = 2048


def _mm_body(x_ref, w_ref, d_ref, g_ref, dis_ref, dis2_ref):
    dd = d_ref[...]
    deg = dd[0] + dd[1] + 1.0
    dis = lax.rsqrt(deg)
    dis_ref[...] = dis
    dis2_ref[...] = 1.0 / deg
    y = jnp.dot(x_ref[...], w_ref[...], preferred_element_type=jnp.float32)
    g_ref[...] = dis[:, None] * y


def _matmul_scale1(x_pad, wt, dparts):
    d2 = dparts.reshape(NC, N_PAD)
    return pl.pallas_call(
        _mm_body,
        grid=(N_PAD // _MM_BLK,),
        in_specs=[
            pl.BlockSpec((_MM_BLK, D_FEAT), lambda i: (i, 0)),
            pl.BlockSpec((D_FEAT, C), lambda i: (0, 0)),
            pl.BlockSpec((NC, _MM_BLK), lambda i: (0, i)),
        ],
        out_specs=[
            pl.BlockSpec((_MM_BLK, C), lambda i: (i, 0)),
            pl.BlockSpec((_MM_BLK,), lambda i: (i,)),
            pl.BlockSpec((_MM_BLK,), lambda i: (i,)),
        ],
        out_shape=[jax.ShapeDtypeStruct((N_PAD, C), jnp.float32),
                   jax.ShapeDtypeStruct((N_PAD,), jnp.float32),
                   jax.ShapeDtypeStruct((N_PAD,), jnp.float32)],
    )(x_pad, wt, d2)


_HG = E_PAD // NW // 16


def _deg_body(z1_hbm, dst2_hbm, out_hbm, SH, dstv, hist, acc, tmp):
    c = lax.axis_index("c")
    s = lax.axis_index("s")
    wid = _worker_id()
    pltpu.sync_copy(dst2_hbm.at[wid], dstv)
    pltpu.sync_copy(z1_hbm, hist)
    ones16 = jnp.ones((16,), jnp.float32)

    def step(j, carry):
        plsc.addupdate_scatter(hist, [dstv[j]], ones16)
        return carry

    lax.fori_loop(0, _HG, step, 0)
    pltpu.sync_copy(hist, SH.at[s])
    plsc.subcore_barrier()
    sl = pl.ds(s * RPS, RPS)
    pltpu.sync_copy(SH.at[0, sl], acc)

    def comb(t, carry):
        pltpu.sync_copy(SH.at[t, sl], tmp)

        def addk(k, carry2):
            kk = pl.ds(k * 16, 16)
            acc[kk] = acc[kk] + tmp[kk]
            return carry2

        lax.fori_loop(0, RPS // 16, addk, 0)
        return carry

    lax.fori_loop(1, NS, comb, 0)
    pltpu.sync_copy(acc, out_hbm.at[pl.ds(c * N_PAD + s * RPS, RPS)])


_prop_deg = pl.kernel(
    _deg_body,
    out_type=jax.ShapeDtypeStruct((NC * N_PAD,), jnp.float32),
    mesh=_MESH,
    scratch_types=[
        pltpu.VMEM_SHARED((NS, N_PAD), jnp.float32),
        pltpu.VMEM((_HG, 16), jnp.int32),
        pltpu.VMEM((N_PAD,), jnp.float32),
        pltpu.VMEM((RPS,), jnp.float32),
        pltpu.VMEM((RPS,), jnp.float32),
    ],
    compiler_params=pltpu.CompilerParams(needs_layout_passes=False),
)


def _make_prop(mode):
    scratch = [
        pltpu.VMEM_SHARED((N_PAD, C), jnp.float32),
        pltpu.VMEM((CH, CHUNK), jnp.int32),
        pltpu.VMEM((K_BUF, CHUNK, C), jnp.float32),
        pltpu.SemaphoreType.DMA((K_BUF,)),
        pltpu.VMEM_SHARED((N_PAD, C), jnp.float32),
        pltpu.VMEM((CH, CHUNK), jnp.int32),
        pltpu.SemaphoreType.DMA((K_BUF,)),
    ]
    if mode == "mid":
        scratch += [pltpu.VMEM((RPS, C), jnp.float32),
                    pltpu.VMEM((RPS, C), jnp.float32),
                    pltpu.VMEM((RPS,), jnp.float32),
                    pltpu.VMEM((RPS, C), jnp.float32)]

    def body(g_hbm, aux_hbm, src_hbm, dst_hbm, z_hbm, out_hbm, S, dstv, rows,
             ssem, *rest):
        c = lax.axis_index("c")
        s = lax.axis_index("s")
        wid = _worker_id()
        pltpu.sync_copy(dst_hbm.at[wid], dstv)
        sl = pl.ds(s * RPS, RPS)
        G, srcv, gsem = rest[:3]
        pltpu.sync_copy(src_hbm.at[wid], srcv)
        if mode == "prop":
            pltpu.sync_copy(g_hbm.at[sl], G.at[sl])
        else:
            p0v, p1v, d2v, gv = rest[3:]
            pltpu.sync_copy(g_hbm.at[pl.ds(s * RPS, RPS)], p0v)
            pltpu.sync_copy(g_hbm.at[pl.ds(N_PAD + s * RPS, RPS)], p1v)
            pltpu.sync_copy(aux_hbm.at[sl], d2v)

            def srow(i, carry):
                dvec = d2v[pl.ds(i * 16, 16)]
                for k in range(16):
                    r = i * 16 + k
                    gv[r] = (p0v[r] + p1v[r]) * dvec[k]
                return carry

            lax.fori_loop(0, RPS // 16, srow, 0)
            pltpu.sync_copy(gv, G.at[sl])

        @pl.when(c == 0)
        def _():
            if mode == "mid":
                pltpu.sync_copy(gv, S.at[sl])
            else:
                pltpu.sync_copy(g_hbm.at[sl], S.at[sl])

        @pl.when(c != 0)
        def _():
            pltpu.sync_copy(z_hbm.at[sl], S.at[sl])

        plsc.subcore_barrier()

        gdesc = [None] * CH
        sdesc = [None] * CH

        def issue_scatter(j):
            gdesc[j].wait()
            sdesc[j] = pltpu.async_copy(
                rows.at[j % K_BUF], S.at[dstv.at[j]],
                ssem.at[j % K_BUF], add=True)

        for j in range(CH):
            if j >= K_BUF:
                sdesc[j - K_BUF].wait()
            gdesc[j] = pltpu.async_copy(G.at[srcv.at[j]],
                                        rows.at[j % K_BUF],
                                        gsem.at[j % K_BUF])
            if j >= LAG:
                issue_scatter(j - LAG)
        for j in range(CH - LAG, CH):
            issue_scatter(j)
        for j in range(CH - K_BUF, CH):
            sdesc[j].wait()
        plsc.subcore_barrier()
        pltpu.sync_copy(S.at[sl], out_hbm.at[pl.ds(c * N_PAD + s * RPS, RPS)])

    return pl.kernel(
        body,
        out_type=jax.ShapeDtypeStruct((NC * N_PAD, C), jnp.float32),
        mesh=_MESH,
        scratch_types=scratch,
        compiler_params=pltpu.CompilerParams(use_tc_tiling_on_sc=False),
    )


_prop_gather = _make_prop("prop")
_prop_mid = _make_prop("mid")


def _scale3_body(parts_ref, dis_ref, b_ref, o_ref):
    s = parts_ref[:N_PAD, :] + parts_ref[N_PAD:, :]
    o_ref[...] = dis_ref[...][:, None] * s + b_ref[...]


def _scale3(parts, dis, b):
    return pl.pallas_call(
        _scale3_body,
        out_shape=jax.ShapeDtypeStruct((N_PAD, C), jnp.float32),
    )(parts, dis, b.reshape(1, C))


def kernel(x, edge_index, W, b):
    src = edge_index[0].astype(jnp.int32)
    dst = edge_index[1].astype(jnp.int32)
    pad = E_PAD - E
    dummy = jnp.full((pad,), N_NODES, jnp.int32)
    src3 = jnp.concatenate([src, dummy]).reshape(NW, CH, CHUNK)
    dst3 = jnp.concatenate([dst, dummy]).reshape(NW, CH, CHUNK)

    dst2 = jnp.concatenate([dst, dummy]).reshape(NW, _HG, 16)
    x_pad = jnp.pad(x, ((0, N_PAD - N_NODES), (0, 0)))

    zeros1 = jnp.zeros((N_PAD,), jnp.float32)
    zeros_tab = jnp.zeros((N_PAD, C), jnp.float32)

    deg_parts = _prop_deg(zeros1, dst2)
    g1, dis, dis2 = _matmul_scale1(x_pad, W.T, deg_parts)
    p1 = _prop_gather(g1, dis2, src3, dst3, zeros_tab)
    p2 = _prop_mid(p1, dis2, src3, dst3, zeros_tab)
    out_pad = _scale3(p2, dis, b)
    return out_pad[:N_NODES]


# speedup vs baseline: 1.4704x; 1.1685x over previous
import functools

import jax
import jax.numpy as jnp
from jax import lax
from jax.experimental import pallas as pl
from jax.experimental.pallas import tpu as pltpu
from jax.experimental.pallas import tpu_sc as plsc

N_NODES = 10000
D_FEAT = 128
C = 16
NC = 2
NS = 16
NW = NC * NS
N_PAD = 10240
RPS = N_PAD // NS
E = 320000
CHUNK = 128
K_BUF = 8
LAG = 4
CH = 80
E_PAD = NW * CH * CHUNK
_HG = E_PAD // NW // 16

_MESH = plsc.VectorSubcoreMesh(core_axis_name="c", subcore_axis_name="s")


_MM_BLK = 2048


def _mm_body(x_ref, w_ref, y_ref):
    y_ref[...] = jnp.dot(x_ref[...], w_ref[...],
                         preferred_element_type=jnp.float32)


def _matmul(x_pad, wt):
    return pl.pallas_call(
        _mm_body,
        grid=(N_PAD // _MM_BLK,),
        in_specs=[
            pl.BlockSpec((_MM_BLK, D_FEAT), lambda i: (i, 0)),
            pl.BlockSpec((D_FEAT, C), lambda i: (0, 0)),
        ],
        out_specs=pl.BlockSpec((_MM_BLK, C), lambda i: (i, 0)),
        out_shape=jax.ShapeDtypeStruct((N_PAD, C), jnp.float32),
    )(x_pad, wt)


def _rsqrt16(x):
    i = plsc.bitcast(x, jnp.int32)
    i = jnp.int32(0x5F3759DF) - lax.shift_right_arithmetic(i, 1)
    y = plsc.bitcast(i, jnp.float32)
    for _ in range(3):
        y = y * (1.5 - 0.5 * x * y * y)
    return y


def _sgc_body(y_hbm, src_hbm, dst_hbm, z1_hbm, zt_hbm, b_hbm,
              out_hbm, degx_hbm, px_hbm,
              S, G, SHT, dstv, srcv, rows, hist, tmp2d,
              gv, p0v, p1v, disv, dis2v, accv, bv,
              gsem, ssem, hsem, xsem):
    c = lax.axis_index("c")
    s = lax.axis_index("s")
    wid = s * NC + c
    sl = pl.ds(s * RPS, RPS)

    def global_barrier():
        plsc.subcore_barrier()

        @pl.when(s == 0)
        def _():
            pltpu.semaphore_signal(xsem, 1, core_index=1 - c)
            pl.semaphore_wait(xsem, 1)

        plsc.subcore_barrier()

    pltpu.sync_copy(dst_hbm.at[wid], dstv)
    pltpu.sync_copy(src_hbm.at[wid], srcv)
    pltpu.sync_copy(b_hbm, bv)

    pltpu.sync_copy(z1_hbm, hist)
    ones16 = jnp.ones((16,), jnp.float32)

    def hstep(j, carry):
        for t in range(CHUNK // 16):
            idx = dstv[j, pl.ds(t * 16, 16)]
            plsc.addupdate_scatter(hist, [idx], ones16)
        return carry

    lax.fori_loop(0, CH, hstep, 0)
    hd = [pltpu.async_copy(hist.at[pl.ds(t * RPS, RPS)], SHT.at[t, s], hsem)
          for t in range(NS)]
    for d in hd:
        d.wait()
    plsc.subcore_barrier()
    pltpu.sync_copy(SHT.at[s], tmp2d)
    ng = RPS // 16

    def dstep(k, carry):
        kk = pl.ds(k * 16, 16)
        v = tmp2d[0, kk]
        for t in range(1, NS):
            v = v + tmp2d[t, kk]
        accv[kk] = v
        return carry

    lax.fori_loop(0, ng, dstep, 0)
    pltpu.sync_copy(accv, degx_hbm.at[c, s])
    global_barrier()

    pltpu.sync_copy(degx_hbm.at[0, s], disv)
    pltpu.sync_copy(degx_hbm.at[1, s], dis2v)
    pltpu.sync_copy(y_hbm.at[sl], p0v)

    def sc1(k, carry):
        kk = pl.ds(k * 16, 16)
        deg = disv[kk] + dis2v[kk] + 1.0
        dis = _rsqrt16(deg)
        disv[kk] = dis
        dis2v[kk] = 1.0 / deg
        for t in range(16):
            r = k * 16 + t
            gv[r] = p0v[r] * dis[t]
        return carry

    lax.fori_loop(0, ng, sc1, 0)
    pltpu.sync_copy(gv, G.at[sl])

    @pl.when(c == 0)
    def _():
        pltpu.sync_copy(gv, S.at[sl])

    @pl.when(c != 0)
    def _():
        pltpu.sync_copy(zt_hbm.at[sl], S.at[sl])

    plsc.subcore_barrier()

    def edge_loop():
        gdesc = [None] * CH
        sdesc = [None] * CH

        def issue_scatter(j):
            gdesc[j].wait()
            sdesc[j] = pltpu.async_copy(
                rows.at[j % K_BUF], S.at[dstv.at[j]],
                ssem.at[j % K_BUF], add=True)

        for j in range(CH):
            if j >= K_BUF:
                sdesc[j - K_BUF].wait()
            gdesc[j] = pltpu.async_copy(G.at[srcv.at[j]],
                                        rows.at[j % K_BUF],
                                        gsem.at[j % K_BUF])
            if j >= LAG:
                issue_scatter(j - LAG)
        for j in range(CH - LAG, CH):
            issue_scatter(j)
        for j in range(CH - K_BUF, CH):
            sdesc[j].wait()

    edge_loop()
    plsc.subcore_barrier()
    pltpu.sync_copy(S.at[sl], px_hbm.at[c].at[sl])
    global_barrier()

    pltpu.sync_copy(px_hbm.at[0].at[sl], p0v)
    pltpu.sync_copy(px_hbm.at[1].at[sl], p1v)

    def sc2(k, carry):
        d2 = dis2v[pl.ds(k * 16, 16)]
        for t in range(16):
            r = k * 16 + t
            gv[r] = (p0v[r] + p1v[r]) * d2[t]
        return carry

    lax.fori_loop(0, ng, sc2, 0)
    pltpu.sync_copy(gv, G.at[sl])

    @pl.when(c == 0)
    def _():
        pltpu.sync_copy(gv, S.at[sl])

    @pl.when(c != 0)
    def _():
        pltpu.sync_copy(zt_hbm.at[sl], S.at[sl])

    global_barrier()

    edge_loop()
    plsc.subcore_barrier()
    pltpu.sync_copy(S.at[sl], px_hbm.at[c].at[sl])
    global_barrier()

    @pl.when(c == 0)
    def _():
        pltpu.sync_copy(px_hbm.at[0].at[sl], p0v)
        pltpu.sync_copy(px_hbm.at[1].at[sl], p1v)
        bb = bv[...]

        def sc3(k, carry):
            dis = disv[pl.ds(k * 16, 16)]
            for t in range(16):
                r = k * 16 + t
                gv[r] = (p0v[r] + p1v[r]) * dis[t] + bb
            return carry

        lax.fori_loop(0, ng, sc3, 0)
        pltpu.sync_copy(gv, out_hbm.at[sl])


_sgc = pl.kernel(
    _sgc_body,
    out_type=(jax.ShapeDtypeStruct((N_PAD, C), jnp.float32),
              jax.ShapeDtypeStruct((NC, NS, RPS), jnp.float32),
              jax.ShapeDtypeStruct((NC, N_PAD, C), jnp.float32)),
    mesh=_MESH,
    scratch_types=[
        pltpu.VMEM_SHARED((N_PAD, C), jnp.float32),
        pltpu.VMEM_SHARED((N_PAD, C), jnp.float32),
        pltpu.VMEM_SHARED((NS, NS, RPS), jnp.float32),
        pltpu.VMEM((CH, CHUNK), jnp.int32),
        pltpu.VMEM((CH, CHUNK), jnp.int32),
        pltpu.VMEM((K_BUF, CHUNK, C), jnp.float32),
        pltpu.VMEM((N_PAD,), jnp.float32),
        pltpu.VMEM((NS, RPS), jnp.float32),
        pltpu.VMEM((RPS, C), jnp.float32),
        pltpu.VMEM((RPS, C), jnp.float32),
        pltpu.VMEM((RPS, C), jnp.float32),
        pltpu.VMEM((RPS,), jnp.float32),
        pltpu.VMEM((RPS,), jnp.float32),
        pltpu.VMEM((RPS,), jnp.float32),
        pltpu.VMEM((C,), jnp.float32),
        pltpu.SemaphoreType.DMA((K_BUF,)),
        pltpu.SemaphoreType.DMA((K_BUF,)),
        pltpu.SemaphoreType.DMA,
        pltpu.SemaphoreType.REGULAR,
    ],
    compiler_params=pltpu.CompilerParams(use_tc_tiling_on_sc=False,
                                         needs_layout_passes=False),
)


def kernel(x, edge_index, W, b):
    src = edge_index[0].astype(jnp.int32)
    dst = edge_index[1].astype(jnp.int32)
    pad = E_PAD - E
    dummy = jnp.full((pad,), N_NODES, jnp.int32)
    src3 = jnp.concatenate([src, dummy]).reshape(NW, CH, CHUNK)
    dst3 = jnp.concatenate([dst, dummy]).reshape(NW, CH, CHUNK)
    x_pad = jnp.pad(x, ((0, N_PAD - N_NODES), (0, 0)))

    zeros1 = jnp.zeros((N_PAD,), jnp.float32)
    zeros_tab = jnp.zeros((N_PAD, C), jnp.float32)

    y = _matmul(x_pad, W.T)
    out_pad, _, _ = _sgc(y, src3, dst3, zeros1, zeros_tab, b)
    return out_pad[:N_NODES]
